# baseline (speedup 1.0000x reference)
import functools

import jax
import jax.numpy as jnp
from jax import lax
from jax.experimental import pallas as pl
from jax.experimental.pallas import tpu as pltpu
from jax.experimental.pallas import tpu_sc as plsc

NN = 10000
EE = 320000
DIN = 128
DH = 256
DOUT = 64
NG = 16
NP = 10240
HALF = DH // 2

BN = 1024
NBLK = NP // BN

RPT = NP // 16
DPCH = 125
HPT = EE // 32
HNCH = HPT // DPCH

PCH = 25
EPT = EE // 16
PNCH = EPT // PCH
STG = 40
NSTG = PNCH // STG
NBUF = 8

_mesh = plsc.VectorSubcoreMesh(core_axis_name="c", subcore_axis_name="s")


@functools.partial(
    pl.kernel,
    mesh=_mesh,
    out_type=jax.ShapeDtypeStruct((2, NP), jnp.float32),
    scratch_types=[
        pltpu.VMEM((HNCH, DPCH), jnp.int32),
        pltpu.VMEM((DPCH,), jnp.float32),
        pltpu.VMEM_SHARED((NP,), jnp.float32),
        pltpu.SemaphoreType.DMA,
    ],
)
def _sc_degree(dst_hbm, ones_hbm, zeros_hbm, deg_out, dst_v, ones_v, deg_sp,
               sem):
    c = lax.axis_index("c")
    s = lax.axis_index("s")
    w = c * 16 + s
    r0 = s * RPT
    pltpu.sync_copy(zeros_hbm, deg_sp.at[pl.ds(r0, RPT)])
    pltpu.sync_copy(dst_hbm.at[w], dst_v)
    pltpu.sync_copy(ones_hbm, ones_v)
    plsc.subcore_barrier()

    def grp(g, carry):
        for k in range(8):
            pltpu.async_copy(ones_v, deg_sp.at[dst_v.at[g * 8 + k]], sem,
                             add=True)
        for k in range(8):
            pltpu.make_async_copy(ones_v, deg_sp.at[dst_v.at[0]], sem).wait()
        return carry

    lax.fori_loop(0, HNCH // 8, grp, 0)
    plsc.subcore_barrier()
    pltpu.sync_copy(deg_sp.at[pl.ds(r0, RPT)], deg_out.at[c, pl.ds(r0, RPT)])


@functools.partial(
    pl.kernel,
    mesh=_mesh,
    out_type=jax.ShapeDtypeStruct((2, NP, HALF), jnp.float32),
    scratch_types=[
        pltpu.VMEM((STG, PCH), jnp.int32),
        pltpu.VMEM((STG, PCH), jnp.int32),
        pltpu.VMEM((PCH, HALF), jnp.float32),
        pltpu.VMEM((PCH, HALF), jnp.float32),
        pltpu.VMEM((PCH, HALF), jnp.float32),
        pltpu.VMEM((PCH, HALF), jnp.float32),
        pltpu.VMEM((PCH, HALF), jnp.float32),
        pltpu.VMEM((PCH, HALF), jnp.float32),
        pltpu.VMEM((PCH, HALF), jnp.float32),
        pltpu.VMEM((PCH, HALF), jnp.float32),
        pltpu.VMEM_SHARED((NP, HALF), jnp.float32),
        pltpu.SemaphoreType.DMA,
        pltpu.SemaphoreType.DMA,
        pltpu.SemaphoreType.DMA,
        pltpu.SemaphoreType.DMA,
        pltpu.SemaphoreType.DMA,
        pltpu.SemaphoreType.DMA,
        pltpu.SemaphoreType.DMA,
        pltpu.SemaphoreType.DMA,
    ],
)
def _sc_propagate(src_hbm, dst_hbm, g_hbm, zeros_hbm, u_out,
                  src_v, dst_v, rows0, rows1, rows2, rows3, rows4, rows5,
                  rows6, rows7, acc_sp,
                  sem0, sem1, sem2, sem3, sem4, sem5, sem6, sem7):
    c = lax.axis_index("c")
    s = lax.axis_index("s")
    r0 = s * RPT
    rows = (rows0, rows1, rows2, rows3, rows4, rows5, rows6, rows7)
    sems = (sem0, sem1, sem2, sem3, sem4, sem5, sem6, sem7)
    pltpu.sync_copy(zeros_hbm, acc_sp.at[pl.ds(r0, RPT)])
    plsc.subcore_barrier()
    gc = g_hbm.at[c]
    src_t = src_hbm.at[s]
    dst_t = dst_hbm.at[s]

    def stage(st, carry):
        pltpu.sync_copy(src_t.at[pl.ds(st * STG, STG)], src_v)
        pltpu.sync_copy(dst_t.at[pl.ds(st * STG, STG)], dst_v)
        for i in range(NBUF):
            pltpu.async_copy(gc.at[src_v.at[i]], rows[i], sems[i])

        def grp(p, inner):
            for i in range(NBUF):
                pltpu.make_async_copy(gc.at[src_v.at[0]], rows[i],
                                      sems[i]).wait()
                pltpu.async_copy(rows[i],
                                 acc_sp.at[dst_v.at[NBUF * p + i]],
                                 sems[i], add=True)
            for i in range(NBUF):
                pltpu.make_async_copy(rows[i], acc_sp.at[dst_v.at[0]],
                                      sems[i]).wait()
                pltpu.async_copy(gc.at[src_v.at[NBUF * (p + 1) + i]],
                                 rows[i], sems[i])
            return inner

        lax.fori_loop(0, STG // NBUF - 1, grp, 0)
        for i in range(NBUF):
            pltpu.make_async_copy(gc.at[src_v.at[0]], rows[i],
                                  sems[i]).wait()
            pltpu.async_copy(rows[i], acc_sp.at[dst_v.at[STG - NBUF + i]],
                             sems[i], add=True)
        for i in range(NBUF):
            pltpu.make_async_copy(rows[i], acc_sp.at[dst_v.at[0]],
                                  sems[i]).wait()
        return carry

    lax.fori_loop(0, NSTG, stage, 0)
    plsc.subcore_barrier()
    pltpu.sync_copy(acc_sp.at[pl.ds(r0, RPT)], u_out.at[c, pl.ds(r0, RPT)])


def _dis(deg_ref):
    return lax.rsqrt(deg_ref[0] + deg_ref[1] + 1.0)


def _k1_body(x_ref, w_ref, deg_ref, g_ref):
    dis = _dis(deg_ref)
    h = jnp.dot(x_ref[...], w_ref[...], preferred_element_type=jnp.float32)
    g = h * dis[:, None]
    g_ref[0] = g[:, :HALF]
    g_ref[1] = g[:, HALF:]


def _mid_body(u_ref, gp_ref, deg_ref, b_ref, w_ref, g_ref):
    dis = _dis(deg_ref)
    u = jnp.concatenate([u_ref[0], u_ref[1]], axis=1)
    gp = jnp.concatenate([gp_ref[0], gp_ref[1]], axis=1)
    z = jnp.maximum(dis[:, None] * (u + gp) + b_ref[...], 0.0)
    h = jnp.dot(z, w_ref[...], preferred_element_type=jnp.float32)
    g = h * dis[:, None]
    g_ref[0] = g[:, :HALF]
    g_ref[1] = g[:, HALF:]


def _pool_body(u_ref, gp_ref, deg_ref, b_ref, batch_ref, wl_ref, bl_ref,
               out_ref, sums, cnt):
    b = pl.program_id(0)
    dis = _dis(deg_ref)
    u = jnp.concatenate([u_ref[0], u_ref[1]], axis=1)
    gp = jnp.concatenate([gp_ref[0], gp_ref[1]], axis=1)
    z = dis[:, None] * (u + gp) + b_ref[...]
    gids = batch_ref[0]
    oh = (gids[:, None] == lax.broadcasted_iota(jnp.int32, (1, NG), 1)
          ).astype(jnp.float32)
    part = lax.dot_general(oh, z, (((0,), (0,)), ((), ())),
                           preferred_element_type=jnp.float32)
    cpart = jnp.sum(oh, axis=0)[None, :]

    @pl.when(b == 0)
    def _():
        sums[...] = part
        cnt[...] = cpart

    @pl.when(b > 0)
    def _():
        sums[...] += part
        cnt[...] += cpart

    @pl.when(b == NBLK - 1)
    def _():
        pooled = sums[...] / jnp.maximum(cnt[...][0][:, None], 1.0)
        out_ref[...] = (jnp.dot(pooled, wl_ref[...],
                                preferred_element_type=jnp.float32)
                        + bl_ref[...])


def _run_k1(xp, W1, deg_part):
    return pl.pallas_call(
        _k1_body,
        grid=(NBLK,),
        in_specs=[
            pl.BlockSpec((BN, DIN), lambda b: (b, 0)),
            pl.BlockSpec((DIN, DH), lambda b: (0, 0)),
            pl.BlockSpec((2, BN), lambda b: (0, b)),
        ],
        out_specs=pl.BlockSpec((2, BN, HALF), lambda b: (0, b, 0)),
        out_shape=jax.ShapeDtypeStruct((2, NP, HALF), jnp.float32),
    )(xp, W1, deg_part)


def _run_mid(u, gp, deg_part, bias, W):
    return pl.pallas_call(
        _mid_body,
        grid=(NBLK,),
        in_specs=[
            pl.BlockSpec((2, BN, HALF), lambda b: (0, b, 0)),
            pl.BlockSpec((2, BN, HALF), lambda b: (0, b, 0)),
            pl.BlockSpec((2, BN), lambda b: (0, b)),
            pl.BlockSpec((1, DH), lambda b: (0, 0)),
            pl.BlockSpec((DH, DH), lambda b: (0, 0)),
        ],
        out_specs=pl.BlockSpec((2, BN, HALF), lambda b: (0, b, 0)),
        out_shape=jax.ShapeDtypeStruct((2, NP, HALF), jnp.float32),
    )(u, gp, deg_part, bias, W)


def _run_pool(u, gp, deg_part, bias, batch_p, Wlin, blin):
    return pl.pallas_call(
        _pool_body,
        grid=(NBLK,),
        in_specs=[
            pl.BlockSpec((2, BN, HALF), lambda b: (0, b, 0)),
            pl.BlockSpec((2, BN, HALF), lambda b: (0, b, 0)),
            pl.BlockSpec((2, BN), lambda b: (0, b)),
            pl.BlockSpec((1, DH), lambda b: (0, 0)),
            pl.BlockSpec((1, BN), lambda b: (0, b)),
            pl.BlockSpec((DH, DOUT), lambda b: (0, 0)),
            pl.BlockSpec((1, DOUT), lambda b: (0, 0)),
        ],
        out_specs=pl.BlockSpec((NG, DOUT), lambda b: (0, 0)),
        out_shape=jax.ShapeDtypeStruct((NG, DOUT), jnp.float32),
        scratch_shapes=[
            pltpu.VMEM((NG, DH), jnp.float32),
            pltpu.VMEM((1, NG), jnp.float32),
        ],
    )(u, gp, deg_part, bias, batch_p, Wlin, blin)


def kernel(x, edge_index, batch, W1, b1, W2, b2, W3, b3, Wlin, blin):
    src = edge_index[0]
    dst = edge_index[1]
    xp = jnp.zeros((NP, DIN), jnp.float32).at[:NN].set(x)
    batch_p = jnp.full((1, NP), NG, jnp.int32).at[0, :NN].set(batch)
    dst_hist = dst.reshape(32, HNCH, DPCH)
    src_prop = src.reshape(16, PNCH, PCH)
    dst_prop = dst.reshape(16, PNCH, PCH)
    ones_chunk = jnp.ones((DPCH,), jnp.float32)
    zeros_deg = jnp.zeros((RPT,), jnp.float32)
    zeros_rows = jnp.zeros((RPT, HALF), jnp.float32)

    deg_part = _sc_degree(dst_hist, ones_chunk, zeros_deg)

    g1 = _run_k1(xp, W1, deg_part)
    u1 = _sc_propagate(src_prop, dst_prop, g1, zeros_rows)
    g2 = _run_mid(u1, g1, deg_part, b1.reshape(1, DH), W2)
    u2 = _sc_propagate(src_prop, dst_prop, g2, zeros_rows)
    g3 = _run_mid(u2, g2, deg_part, b2.reshape(1, DH), W3)
    u3 = _sc_propagate(src_prop, dst_prop, g3, zeros_rows)
    return _run_pool(u3, g3, deg_part, b3.reshape(1, DH), batch_p,
                     Wlin, blin.reshape(1, DOUT))


# speedup vs baseline: 1.1310x; 1.1310x over previous
import functools

import jax
import jax.numpy as jnp
from jax import lax
from jax.experimental import pallas as pl
from jax.experimental.pallas import tpu as pltpu
from jax.experimental.pallas import tpu_sc as plsc

NN = 10000
EE = 320000
DIN = 128
DH = 256
DOUT = 64
NG = 16
NP = 10240
HALF = DH // 2

BN = 1024
NBLK = NP // BN

RPT = NP // 16
DPCH = 125
HPT = EE // 32
HNCH = HPT // DPCH

PCH = 50
EPT = EE // 16
PNCH = EPT // PCH
STG = 20
NSTG = PNCH // STG
NBUF = 5

_mesh = plsc.VectorSubcoreMesh(core_axis_name="c", subcore_axis_name="s")


@functools.partial(
    pl.kernel,
    mesh=_mesh,
    out_type=jax.ShapeDtypeStruct((2, NP), jnp.float32),
    scratch_types=[
        pltpu.VMEM((HNCH, DPCH), jnp.int32),
        pltpu.VMEM((DPCH,), jnp.float32),
        pltpu.VMEM_SHARED((NP,), jnp.float32),
        pltpu.SemaphoreType.DMA,
    ],
)
def _sc_degree(dst_hbm, ones_hbm, zeros_hbm, deg_out, dst_v, ones_v, deg_sp,
               sem):
    c = lax.axis_index("c")
    s = lax.axis_index("s")
    w = c * 16 + s
    r0 = s * RPT
    pltpu.sync_copy(zeros_hbm, deg_sp.at[pl.ds(r0, RPT)])
    pltpu.sync_copy(dst_hbm.at[w], dst_v)
    pltpu.sync_copy(ones_hbm, ones_v)
    plsc.subcore_barrier()

    def grp(g, carry):
        for k in range(8):
            pltpu.async_copy(ones_v, deg_sp.at[dst_v.at[g * 8 + k]], sem,
                             add=True)
        for k in range(8):
            pltpu.make_async_copy(ones_v, deg_sp.at[dst_v.at[0]], sem).wait()
        return carry

    lax.fori_loop(0, HNCH // 8, grp, 0)
    plsc.subcore_barrier()
    pltpu.sync_copy(deg_sp.at[pl.ds(r0, RPT)], deg_out.at[c, pl.ds(r0, RPT)])


@functools.partial(
    pl.kernel,
    mesh=_mesh,
    out_type=jax.ShapeDtypeStruct((2, NP, HALF), jnp.float32),
    scratch_types=[
        pltpu.VMEM((STG, PCH), jnp.int32),
        pltpu.VMEM((STG, PCH), jnp.int32),
        pltpu.VMEM((STG, PCH), jnp.int32),
        pltpu.VMEM((STG, PCH), jnp.int32),
        pltpu.VMEM((PCH, HALF), jnp.float32),
        pltpu.VMEM((PCH, HALF), jnp.float32),
        pltpu.VMEM((PCH, HALF), jnp.float32),
        pltpu.VMEM((PCH, HALF), jnp.float32),
        pltpu.VMEM((PCH, HALF), jnp.float32),
        pltpu.VMEM_SHARED((NP, HALF), jnp.float32),
        pltpu.SemaphoreType.DMA,
        pltpu.SemaphoreType.DMA,
        pltpu.SemaphoreType.DMA,
        pltpu.SemaphoreType.DMA,
        pltpu.SemaphoreType.DMA,
        pltpu.SemaphoreType.DMA,
    ],
)
def _sc_propagate(src_hbm, dst_hbm, g_hbm, zeros_hbm, u_out,
                  src_a, dst_a, src_b, dst_b,
                  rows0, rows1, rows2, rows3, rows4, acc_sp,
                  sem0, sem1, sem2, sem3, sem4, sem_i):
    c = lax.axis_index("c")
    s = lax.axis_index("s")
    r0 = s * RPT
    rows = (rows0, rows1, rows2, rows3, rows4)
    sems = (sem0, sem1, sem2, sem3, sem4)
    pltpu.sync_copy(zeros_hbm, acc_sp.at[pl.ds(r0, RPT)])
    plsc.subcore_barrier()
    gc = g_hbm.at[c]
    src_t = src_hbm.at[s]
    dst_t = dst_hbm.at[s]

    def run_stage(src_v, dst_v):
        for i in range(NBUF):
            pltpu.async_copy(gc.at[src_v.at[i]], rows[i], sems[i])

        def grp(p, inner):
            for i in range(NBUF):
                pltpu.make_async_copy(gc.at[src_v.at[0]], rows[i],
                                      sems[i]).wait()
                pltpu.async_copy(rows[i],
                                 acc_sp.at[dst_v.at[NBUF * p + i]],
                                 sems[i], add=True)
            for i in range(NBUF):
                pltpu.make_async_copy(rows[i], acc_sp.at[dst_v.at[0]],
                                      sems[i]).wait()
                pltpu.async_copy(gc.at[src_v.at[NBUF * (p + 1) + i]],
                                 rows[i], sems[i])
            return inner

        lax.fori_loop(0, STG // NBUF - 1, grp, 0)
        for i in range(NBUF):
            pltpu.make_async_copy(gc.at[src_v.at[0]], rows[i],
                                  sems[i]).wait()
            pltpu.async_copy(rows[i], acc_sp.at[dst_v.at[STG - NBUF + i]],
                             sems[i], add=True)
        for i in range(NBUF):
            pltpu.make_async_copy(rows[i], acc_sp.at[dst_v.at[0]],
                                  sems[i]).wait()

    def prefetch(st, sv, dv):
        pltpu.async_copy(src_t.at[st], sv, sem_i)
        pltpu.async_copy(dst_t.at[st], dv, sem_i)

    def wait_prefetch(sv, dv):
        pltpu.make_async_copy(src_t.at[0], sv, sem_i).wait()
        pltpu.make_async_copy(dst_t.at[0], dv, sem_i).wait()

    pltpu.sync_copy(src_t.at[0], src_a)
    pltpu.sync_copy(dst_t.at[0], dst_a)

    def dbl(t, carry):
        prefetch(2 * t + 1, src_b, dst_b)
        run_stage(src_a, dst_a)
        wait_prefetch(src_b, dst_b)
        prefetch(lax.rem(2 * t + 2, NSTG), src_a, dst_a)
        run_stage(src_b, dst_b)
        wait_prefetch(src_a, dst_a)
        return carry

    lax.fori_loop(0, NSTG // 2, dbl, 0)
    plsc.subcore_barrier()
    pltpu.sync_copy(acc_sp.at[pl.ds(r0, RPT)], u_out.at[c, pl.ds(r0, RPT)])


def _dis(deg_ref):
    return lax.rsqrt(deg_ref[0] + deg_ref[1] + 1.0)


def _k1_body(x_ref, w_ref, deg_ref, g_ref):
    dis = _dis(deg_ref)
    h = jnp.dot(x_ref[...], w_ref[...], preferred_element_type=jnp.float32)
    g = h * dis[:, None]
    g_ref[0] = g[:, :HALF]
    g_ref[1] = g[:, HALF:]


def _mid_body(u_ref, gp_ref, deg_ref, b_ref, w_ref, g_ref):
    dis = _dis(deg_ref)
    u = jnp.concatenate([u_ref[0], u_ref[1]], axis=1)
    gp = jnp.concatenate([gp_ref[0], gp_ref[1]], axis=1)
    z = jnp.maximum(dis[:, None] * (u + gp) + b_ref[...], 0.0)
    h = jnp.dot(z, w_ref[...], preferred_element_type=jnp.float32)
    g = h * dis[:, None]
    g_ref[0] = g[:, :HALF]
    g_ref[1] = g[:, HALF:]


def _pool_body(u_ref, gp_ref, deg_ref, b_ref, batch_ref, wl_ref, bl_ref,
               out_ref, sums, cnt):
    b = pl.program_id(0)
    dis = _dis(deg_ref)
    u = jnp.concatenate([u_ref[0], u_ref[1]], axis=1)
    gp = jnp.concatenate([gp_ref[0], gp_ref[1]], axis=1)
    z = dis[:, None] * (u + gp) + b_ref[...]
    gids = batch_ref[0]
    oh = (gids[:, None] == lax.broadcasted_iota(jnp.int32, (1, NG), 1)
          ).astype(jnp.float32)
    part = lax.dot_general(oh, z, (((0,), (0,)), ((), ())),
                           preferred_element_type=jnp.float32)
    cpart = jnp.sum(oh, axis=0)[None, :]

    @pl.when(b == 0)
    def _():
        sums[...] = part
        cnt[...] = cpart

    @pl.when(b > 0)
    def _():
        sums[...] += part
        cnt[...] += cpart

    @pl.when(b == NBLK - 1)
    def _():
        pooled = sums[...] / jnp.maximum(cnt[...][0][:, None], 1.0)
        out_ref[...] = (jnp.dot(pooled, wl_ref[...],
                                preferred_element_type=jnp.float32)
                        + bl_ref[...])


def _run_k1(xp, W1, deg_part):
    return pl.pallas_call(
        _k1_body,
        grid=(NBLK,),
        in_specs=[
            pl.BlockSpec((BN, DIN), lambda b: (b, 0)),
            pl.BlockSpec((DIN, DH), lambda b: (0, 0)),
            pl.BlockSpec((2, BN), lambda b: (0, b)),
        ],
        out_specs=pl.BlockSpec((2, BN, HALF), lambda b: (0, b, 0)),
        out_shape=jax.ShapeDtypeStruct((2, NP, HALF), jnp.float32),
    )(xp, W1, deg_part)


def _run_mid(u, gp, deg_part, bias, W):
    return pl.pallas_call(
        _mid_body,
        grid=(NBLK,),
        in_specs=[
            pl.BlockSpec((2, BN, HALF), lambda b: (0, b, 0)),
            pl.BlockSpec((2, BN, HALF), lambda b: (0, b, 0)),
            pl.BlockSpec((2, BN), lambda b: (0, b)),
            pl.BlockSpec((1, DH), lambda b: (0, 0)),
            pl.BlockSpec((DH, DH), lambda b: (0, 0)),
        ],
        out_specs=pl.BlockSpec((2, BN, HALF), lambda b: (0, b, 0)),
        out_shape=jax.ShapeDtypeStruct((2, NP, HALF), jnp.float32),
    )(u, gp, deg_part, bias, W)


def _run_pool(u, gp, deg_part, bias, batch_p, Wlin, blin):
    return pl.pallas_call(
        _pool_body,
        grid=(NBLK,),
        in_specs=[
            pl.BlockSpec((2, BN, HALF), lambda b: (0, b, 0)),
            pl.BlockSpec((2, BN, HALF), lambda b: (0, b, 0)),
            pl.BlockSpec((2, BN), lambda b: (0, b)),
            pl.BlockSpec((1, DH), lambda b: (0, 0)),
            pl.BlockSpec((1, BN), lambda b: (0, b)),
            pl.BlockSpec((DH, DOUT), lambda b: (0, 0)),
            pl.BlockSpec((1, DOUT), lambda b: (0, 0)),
        ],
        out_specs=pl.BlockSpec((NG, DOUT), lambda b: (0, 0)),
        out_shape=jax.ShapeDtypeStruct((NG, DOUT), jnp.float32),
        scratch_shapes=[
            pltpu.VMEM((NG, DH), jnp.float32),
            pltpu.VMEM((1, NG), jnp.float32),
        ],
    )(u, gp, deg_part, bias, batch_p, Wlin, blin)


def kernel(x, edge_index, batch, W1, b1, W2, b2, W3, b3, Wlin, blin):
    src = edge_index[0]
    dst = edge_index[1]
    xp = jnp.zeros((NP, DIN), jnp.float32).at[:NN].set(x)
    batch_p = jnp.full((1, NP), NG, jnp.int32).at[0, :NN].set(batch)
    dst_hist = dst.reshape(32, HNCH, DPCH)
    src_prop = src.reshape(16, NSTG, STG, PCH)
    dst_prop = dst.reshape(16, NSTG, STG, PCH)
    ones_chunk = jnp.ones((DPCH,), jnp.float32)
    zeros_deg = jnp.zeros((RPT,), jnp.float32)
    zeros_rows = jnp.zeros((RPT, HALF), jnp.float32)

    deg_part = _sc_degree(dst_hist, ones_chunk, zeros_deg)

    g1 = _run_k1(xp, W1, deg_part)
    u1 = _sc_propagate(src_prop, dst_prop, g1, zeros_rows)
    g2 = _run_mid(u1, g1, deg_part, b1.reshape(1, DH), W2)
    u2 = _sc_propagate(src_prop, dst_prop, g2, zeros_rows)
    g3 = _run_mid(u2, g2, deg_part, b2.reshape(1, DH), W3)
    u3 = _sc_propagate(src_prop, dst_prop, g3, zeros_rows)
    return _run_pool(u3, g3, deg_part, b3.reshape(1, DH), batch_p,
                     Wlin, blin.reshape(1, DOUT))


# speedup vs baseline: 1.1757x; 1.0396x over previous
import functools

import jax
import jax.numpy as jnp
from jax import lax
from jax.experimental import pallas as pl
from jax.experimental.pallas import tpu as pltpu
from jax.experimental.pallas import tpu_sc as plsc

NN = 10000
EE = 320000
DIN = 128
DH = 256
DOUT = 64
NG = 16
NP = 10240
HALF = DH // 2

BN = 1024
NBLK = NP // BN

RPT = NP // 16
DPCH = 125
HPT = EE // 32
HNCH = HPT // DPCH

PCH = 50
EPT = EE // 16
PNCH = EPT // PCH
STG = 20
NSTG = PNCH // STG
NBUF = 5

_mesh = plsc.VectorSubcoreMesh(core_axis_name="c", subcore_axis_name="s")


@functools.partial(
    pl.kernel,
    mesh=_mesh,
    out_type=jax.ShapeDtypeStruct((2, NP), jnp.float32),
    scratch_types=[
        pltpu.VMEM((HNCH, DPCH), jnp.int32),
        pltpu.VMEM((DPCH,), jnp.float32),
        pltpu.VMEM_SHARED((NP,), jnp.float32),
        pltpu.SemaphoreType.DMA,
    ],
)
def _sc_degree(dst_hbm, ones_hbm, zeros_hbm, deg_out, dst_v, ones_v, deg_sp,
               sem):
    c = lax.axis_index("c")
    s = lax.axis_index("s")
    w = c * 16 + s
    r0 = s * RPT
    pltpu.sync_copy(zeros_hbm, deg_sp.at[pl.ds(r0, RPT)])
    pltpu.sync_copy(dst_hbm.at[w], dst_v)
    pltpu.sync_copy(ones_hbm, ones_v)
    plsc.subcore_barrier()

    def grp(g, carry):
        for k in range(8):
            pltpu.async_copy(ones_v, deg_sp.at[dst_v.at[g * 8 + k]], sem,
                             add=True)
        for k in range(8):
            pltpu.make_async_copy(ones_v, deg_sp.at[dst_v.at[0]], sem).wait()
        return carry

    lax.fori_loop(0, HNCH // 8, grp, 0)
    plsc.subcore_barrier()
    pltpu.sync_copy(deg_sp.at[pl.ds(r0, RPT)], deg_out.at[c, pl.ds(r0, RPT)])


@functools.partial(
    pl.kernel,
    mesh=_mesh,
    out_type=jax.ShapeDtypeStruct((2, NP, HALF), jnp.float32),
    scratch_types=[
        pltpu.VMEM((STG, PCH), jnp.int32),
        pltpu.VMEM((STG, PCH), jnp.int32),
        pltpu.VMEM((STG, PCH), jnp.int32),
        pltpu.VMEM((STG, PCH), jnp.int32),
        pltpu.VMEM((PCH, HALF), jnp.float32),
        pltpu.VMEM((PCH, HALF), jnp.float32),
        pltpu.VMEM((PCH, HALF), jnp.float32),
        pltpu.VMEM((PCH, HALF), jnp.float32),
        pltpu.VMEM((PCH, HALF), jnp.float32),
        pltpu.VMEM_SHARED((NP, HALF), jnp.float32),
        pltpu.SemaphoreType.DMA,
        pltpu.SemaphoreType.DMA,
        pltpu.SemaphoreType.DMA,
        pltpu.SemaphoreType.DMA,
        pltpu.SemaphoreType.DMA,
        pltpu.SemaphoreType.DMA,
    ],
)
def _sc_propagate(src_hbm, dst_hbm, g_hbm, zeros_hbm, u_out,
                  src_a, dst_a, src_b, dst_b,
                  rows0, rows1, rows2, rows3, rows4, acc_sp,
                  sem0, sem1, sem2, sem3, sem4, sem_i):
    c = lax.axis_index("c")
    s = lax.axis_index("s")
    r0 = s * RPT
    rows = (rows0, rows1, rows2, rows3, rows4)
    sems = (sem0, sem1, sem2, sem3, sem4)
    pltpu.sync_copy(zeros_hbm, acc_sp.at[pl.ds(r0, RPT)])
    plsc.subcore_barrier()
    gc = g_hbm.at[c]
    src_t = src_hbm.at[s]
    dst_t = dst_hbm.at[s]

    def prefetch(st, sv, dv):
        pltpu.async_copy(src_t.at[st], sv, sem_i)
        pltpu.async_copy(dst_t.at[st], dv, sem_i)

    def wait_prefetch(sv, dv):
        pltpu.make_async_copy(src_t.at[0], sv, sem_i).wait()
        pltpu.make_async_copy(dst_t.at[0], dv, sem_i).wait()

    def run_stage(src_v, dst_v, src_n, dst_n, nxt):
        def grp(p, inner):
            for i in range(NBUF):
                pltpu.make_async_copy(gc.at[src_v.at[0]], rows[i],
                                      sems[i]).wait()
                pltpu.async_copy(rows[i],
                                 acc_sp.at[dst_v.at[NBUF * p + i]],
                                 sems[i], add=True)
            for i in range(NBUF):
                pltpu.make_async_copy(rows[i], acc_sp.at[dst_v.at[0]],
                                      sems[i]).wait()
                pltpu.async_copy(gc.at[src_v.at[NBUF * (p + 1) + i]],
                                 rows[i], sems[i])
            return inner

        lax.fori_loop(0, STG // NBUF - 1, grp, 0)
        for i in range(NBUF):
            pltpu.make_async_copy(gc.at[src_v.at[0]], rows[i],
                                  sems[i]).wait()
            pltpu.async_copy(rows[i], acc_sp.at[dst_v.at[STG - NBUF + i]],
                             sems[i], add=True)
        wait_prefetch(src_n, dst_n)
        for i in range(NBUF):
            pltpu.make_async_copy(rows[i], acc_sp.at[dst_v.at[0]],
                                  sems[i]).wait()
            pltpu.async_copy(gc.at[src_n.at[i]], rows[i], sems[i])
        prefetch(nxt, src_v, dst_v)

    pltpu.sync_copy(src_t.at[0], src_a)
    pltpu.sync_copy(dst_t.at[0], dst_a)
    for i in range(NBUF):
        pltpu.async_copy(gc.at[src_a.at[i]], rows[i], sems[i])
    prefetch(1, src_b, dst_b)

    def dbl(t, carry):
        run_stage(src_a, dst_a, src_b, dst_b, lax.rem(2 * t + 2, NSTG))
        run_stage(src_b, dst_b, src_a, dst_a, lax.rem(2 * t + 3, NSTG))
        return carry

    lax.fori_loop(0, NSTG // 2, dbl, 0)
    for i in range(NBUF):
        pltpu.make_async_copy(gc.at[src_a.at[0]], rows[i], sems[i]).wait()
    wait_prefetch(src_b, dst_b)
    plsc.subcore_barrier()
    pltpu.sync_copy(acc_sp.at[pl.ds(r0, RPT)], u_out.at[c, pl.ds(r0, RPT)])


def _dis(deg_ref):
    return lax.rsqrt(deg_ref[0] + deg_ref[1] + 1.0)


def _k1_body(x_ref, w_ref, deg_ref, g_ref):
    dis = _dis(deg_ref)
    h = jnp.dot(x_ref[...], w_ref[...], preferred_element_type=jnp.float32)
    g = h * dis[:, None]
    g_ref[0] = g[:, :HALF]
    g_ref[1] = g[:, HALF:]


def _mid_body(u_ref, gp_ref, deg_ref, b_ref, w_ref, g_ref):
    dis = _dis(deg_ref)
    u = jnp.concatenate([u_ref[0], u_ref[1]], axis=1)
    gp = jnp.concatenate([gp_ref[0], gp_ref[1]], axis=1)
    z = jnp.maximum(dis[:, None] * (u + gp) + b_ref[...], 0.0)
    h = jnp.dot(z, w_ref[...], preferred_element_type=jnp.float32)
    g = h * dis[:, None]
    g_ref[0] = g[:, :HALF]
    g_ref[1] = g[:, HALF:]


def _pool_body(u_ref, gp_ref, deg_ref, b_ref, batch_ref, wl_ref, bl_ref,
               out_ref, sums, cnt):
    b = pl.program_id(0)
    dis = _dis(deg_ref)
    u = jnp.concatenate([u_ref[0], u_ref[1]], axis=1)
    gp = jnp.concatenate([gp_ref[0], gp_ref[1]], axis=1)
    z = dis[:, None] * (u + gp) + b_ref[...]
    gids = batch_ref[0]
    oh = (gids[:, None] == lax.broadcasted_iota(jnp.int32, (1, NG), 1)
          ).astype(jnp.float32)
    part = lax.dot_general(oh, z, (((0,), (0,)), ((), ())),
                           preferred_element_type=jnp.float32)
    cpart = jnp.sum(oh, axis=0)[None, :]

    @pl.when(b == 0)
    def _():
        sums[...] = part
        cnt[...] = cpart

    @pl.when(b > 0)
    def _():
        sums[...] += part
        cnt[...] += cpart

    @pl.when(b == NBLK - 1)
    def _():
        pooled = sums[...] / jnp.maximum(cnt[...][0][:, None], 1.0)
        out_ref[...] = (jnp.dot(pooled, wl_ref[...],
                                preferred_element_type=jnp.float32)
                        + bl_ref[...])


def _run_k1(xp, W1, deg_part):
    return pl.pallas_call(
        _k1_body,
        grid=(NBLK,),
        in_specs=[
            pl.BlockSpec((BN, DIN), lambda b: (b, 0)),
            pl.BlockSpec((DIN, DH), lambda b: (0, 0)),
            pl.BlockSpec((2, BN), lambda b: (0, b)),
        ],
        out_specs=pl.BlockSpec((2, BN, HALF), lambda b: (0, b, 0)),
        out_shape=jax.ShapeDtypeStruct((2, NP, HALF), jnp.float32),
    )(xp, W1, deg_part)


def _run_mid(u, gp, deg_part, bias, W):
    return pl.pallas_call(
        _mid_body,
        grid=(NBLK,),
        in_specs=[
            pl.BlockSpec((2, BN, HALF), lambda b: (0, b, 0)),
            pl.BlockSpec((2, BN, HALF), lambda b: (0, b, 0)),
            pl.BlockSpec((2, BN), lambda b: (0, b)),
            pl.BlockSpec((1, DH), lambda b: (0, 0)),
            pl.BlockSpec((DH, DH), lambda b: (0, 0)),
        ],
        out_specs=pl.BlockSpec((2, BN, HALF), lambda b: (0, b, 0)),
        out_shape=jax.ShapeDtypeStruct((2, NP, HALF), jnp.float32),
    )(u, gp, deg_part, bias, W)


def _run_pool(u, gp, deg_part, bias, batch_p, Wlin, blin):
    return pl.pallas_call(
        _pool_body,
        grid=(NBLK,),
        in_specs=[
            pl.BlockSpec((2, BN, HALF), lambda b: (0, b, 0)),
            pl.BlockSpec((2, BN, HALF), lambda b: (0, b, 0)),
            pl.BlockSpec((2, BN), lambda b: (0, b)),
            pl.BlockSpec((1, DH), lambda b: (0, 0)),
            pl.BlockSpec((1, BN), lambda b: (0, b)),
            pl.BlockSpec((DH, DOUT), lambda b: (0, 0)),
            pl.BlockSpec((1, DOUT), lambda b: (0, 0)),
        ],
        out_specs=pl.BlockSpec((NG, DOUT), lambda b: (0, 0)),
        out_shape=jax.ShapeDtypeStruct((NG, DOUT), jnp.float32),
        scratch_shapes=[
            pltpu.VMEM((NG, DH), jnp.float32),
            pltpu.VMEM((1, NG), jnp.float32),
        ],
    )(u, gp, deg_part, bias, batch_p, Wlin, blin)


def kernel(x, edge_index, batch, W1, b1, W2, b2, W3, b3, Wlin, blin):
    src = edge_index[0]
    dst = edge_index[1]
    xp = jnp.zeros((NP, DIN), jnp.float32).at[:NN].set(x)
    batch_p = jnp.full((1, NP), NG, jnp.int32).at[0, :NN].set(batch)
    dst_hist = dst.reshape(32, HNCH, DPCH)
    src_prop = src.reshape(16, NSTG, STG, PCH)
    dst_prop = dst.reshape(16, NSTG, STG, PCH)
    ones_chunk = jnp.ones((DPCH,), jnp.float32)
    zeros_deg = jnp.zeros((RPT,), jnp.float32)
    zeros_rows = jnp.zeros((RPT, HALF), jnp.float32)

    deg_part = _sc_degree(dst_hist, ones_chunk, zeros_deg)

    g1 = _run_k1(xp, W1, deg_part)
    u1 = _sc_propagate(src_prop, dst_prop, g1, zeros_rows)
    g2 = _run_mid(u1, g1, deg_part, b1.reshape(1, DH), W2)
    u2 = _sc_propagate(src_prop, dst_prop, g2, zeros_rows)
    g3 = _run_mid(u2, g2, deg_part, b2.reshape(1, DH), W3)
    u3 = _sc_propagate(src_prop, dst_prop, g3, zeros_rows)
    return _run_pool(u3, g3, deg_part, b3.reshape(1, DH), batch_p,
                     Wlin, blin.reshape(1, DOUT))
